# Initial kernel scaffold; baseline (speedup 1.0000x reference)
import jax
import jax.numpy as jnp
from jax.experimental import pallas as pl


def kernel(x, rbf, edge_index, weight1, bias1, weight2, bias2, weight3, weight4, bias4, weight5, bias5):
    raise NotImplementedError("write your pallas kernel here")


# speedup vs baseline: 2.9022x; 2.9022x over previous
import functools

import jax
import jax.numpy as jnp
from jax import lax
from jax.experimental import pallas as pl
from jax.experimental.pallas import tpu as pltpu
from jax.experimental.pallas import tpu_sc as plsc

N_NODES = 10000
N_EDGES = 320000
C = 128
N_RBF = 200

_LOG2 = 0.6931471805599453

_NC = 2
_NS = 16
_NW = _NC * _NS
_EPW = N_EDGES // _NW
_CHUNK = 80
_NCHUNKS = _EPW // _CHUNK
_NPAD = 10240
_RPT = _NPAD // _NS

_EB = 2560
_NB = 2000


def _ssp(x):
    safe = jnp.minimum(x, 14.0)
    return jnp.where(x < 14.0, jnp.log(1.0 + jnp.exp(safe)), x) - _LOG2


def _filter_body(rbf_ref, w1_ref, b1_ref, w2_ref, b2_ref, out_ref):
    r = jnp.dot(rbf_ref[...], w1_ref[...], preferred_element_type=jnp.float32)
    r = _ssp(r + b1_ref[...])
    w = jnp.dot(r, w2_ref[...], preferred_element_type=jnp.float32)
    out_ref[...] = _ssp(w + b2_ref[...])


def _edge_filter(rbf, w1, b1, w2, b2):
    grid = (N_EDGES // _EB,)
    return pl.pallas_call(
        _filter_body,
        grid=grid,
        in_specs=[
            pl.BlockSpec((_EB, N_RBF), lambda i: (i, 0)),
            pl.BlockSpec((N_RBF, C), lambda i: (0, 0)),
            pl.BlockSpec((1, C), lambda i: (0, 0)),
            pl.BlockSpec((C, C), lambda i: (0, 0)),
            pl.BlockSpec((1, C), lambda i: (0, 0)),
        ],
        out_specs=pl.BlockSpec((_EB, C), lambda i: (i, 0)),
        out_shape=jax.ShapeDtypeStruct((N_EDGES, C), jnp.float32),
    )(rbf, w1, b1, w2, b2)


def _segment_sum_sc(src_grouped, w_edges, zeros_tab):
    mesh = plsc.VectorSubcoreMesh(core_axis_name="c", subcore_axis_name="s")

    @functools.partial(
        pl.kernel,
        mesh=mesh,
        out_type=jax.ShapeDtypeStruct((_NC, _NPAD, C), jnp.float32),
        scratch_types=[
            pltpu.VMEM((_NCHUNKS, _CHUNK), jnp.int32),
            pltpu.VMEM((_CHUNK, C), jnp.float32),
            pltpu.VMEM_SHARED((_NPAD, C), jnp.float32),
        ],
    )
    def seg(src_hbm, w_hbm, z_hbm, out_hbm, idx_v, rows_v, table_sh):
        cid = lax.axis_index("c")
        sid = lax.axis_index("s")
        wid = cid * _NS + sid

        pltpu.sync_copy(z_hbm.at[pl.ds(sid * _RPT, _RPT)],
                        table_sh.at[pl.ds(sid * _RPT, _RPT)])
        plsc.subcore_barrier()

        pltpu.sync_copy(src_hbm.at[wid], idx_v)

        def body(k, carry):
            base = wid * _EPW + k * _CHUNK
            pltpu.sync_copy(w_hbm.at[pl.ds(base, _CHUNK)], rows_v)
            pltpu.sync_copy(rows_v, table_sh.at[idx_v.at[k]], add=True)
            return carry

        lax.fori_loop(0, _NCHUNKS, body, 0)
        plsc.subcore_barrier()

        pltpu.sync_copy(table_sh.at[pl.ds(sid * _RPT, _RPT)],
                        out_hbm.at[cid, pl.ds(sid * _RPT, _RPT)])

    return seg(src_grouped, w_edges, zeros_tab)


def _output_body(x_ref, p_ref, w3_ref, w4_ref, b4_ref, w5_ref, b5_ref, out_ref):
    xb = x_ref[...]
    f = jnp.dot(xb, w3_ref[...], preferred_element_type=jnp.float32)
    conv = f * (p_ref[0] + p_ref[1])
    y = _ssp(jnp.dot(conv, w4_ref[...], preferred_element_type=jnp.float32)
             + b4_ref[...])
    v = jnp.dot(y, w5_ref[...], preferred_element_type=jnp.float32) + b5_ref[...]
    out_ref[...] = xb + v


def _node_output(x, partials, w3, w4, b4, w5, b5):
    grid = (N_NODES // _NB,)
    return pl.pallas_call(
        _output_body,
        grid=grid,
        in_specs=[
            pl.BlockSpec((_NB, C), lambda i: (i, 0)),
            pl.BlockSpec((_NC, _NB, C), lambda i: (0, i, 0)),
            pl.BlockSpec((C, C), lambda i: (0, 0)),
            pl.BlockSpec((C, C), lambda i: (0, 0)),
            pl.BlockSpec((1, C), lambda i: (0, 0)),
            pl.BlockSpec((C, C), lambda i: (0, 0)),
            pl.BlockSpec((1, C), lambda i: (0, 0)),
        ],
        out_specs=pl.BlockSpec((_NB, C), lambda i: (i, 0)),
        out_shape=jax.ShapeDtypeStruct((N_NODES, C), jnp.float32),
    )(x, partials, w3, w4, b4, w5, b5)


def kernel(x, rbf, edge_index, weight1, bias1, weight2, bias2, weight3,
           weight4, bias4, weight5, bias5):
    b1 = bias1.reshape(1, C)
    b2 = bias2.reshape(1, C)
    b4 = bias4.reshape(1, C)
    b5 = bias5.reshape(1, C)

    w = _edge_filter(rbf, weight1, b1, weight2, b2)

    src = edge_index[0].astype(jnp.int32).reshape(_NW, _NCHUNKS, _CHUNK)
    zeros_tab = jnp.zeros((_NPAD, C), jnp.float32)
    partials = _segment_sum_sc(src, w, zeros_tab)

    return _node_output(x, partials, weight3, weight4, b4, weight5, b5)


# speedup vs baseline: 3.2966x; 1.1359x over previous
import functools

import jax
import jax.numpy as jnp
from jax import lax
from jax.experimental import pallas as pl
from jax.experimental.pallas import tpu as pltpu
from jax.experimental.pallas import tpu_sc as plsc

N_NODES = 10000
N_EDGES = 320000
C = 128
N_RBF = 200

_LOG2 = 0.6931471805599453

_NC = 2
_NS = 16
_NW = _NC * _NS
_EPW = N_EDGES // _NW
_CHUNK = 80
_NCHUNKS = _EPW // _CHUNK
_NPAD = 10240
_RPT = _NPAD // _NS

_EB = 2560
_NB = 2000


def _ssp(x):
    safe = jnp.minimum(x, 14.0)
    return jnp.where(x < 14.0, jnp.log(1.0 + jnp.exp(safe)), x) - _LOG2


def _filter_body(rbf_ref, w1_ref, b1_ref, w2_ref, b2_ref, out_ref):
    r = jnp.dot(rbf_ref[...], w1_ref[...], preferred_element_type=jnp.float32)
    r = _ssp(r + b1_ref[...])
    w = jnp.dot(r, w2_ref[...], preferred_element_type=jnp.float32)
    out_ref[...] = _ssp(w + b2_ref[...])


def _edge_filter(rbf, w1, b1, w2, b2):
    grid = (N_EDGES // _EB,)
    return pl.pallas_call(
        _filter_body,
        grid=grid,
        in_specs=[
            pl.BlockSpec((_EB, N_RBF), lambda i: (i, 0)),
            pl.BlockSpec((N_RBF, C), lambda i: (0, 0)),
            pl.BlockSpec((1, C), lambda i: (0, 0)),
            pl.BlockSpec((C, C), lambda i: (0, 0)),
            pl.BlockSpec((1, C), lambda i: (0, 0)),
        ],
        out_specs=pl.BlockSpec((_EB, C), lambda i: (i, 0)),
        out_shape=jax.ShapeDtypeStruct((N_EDGES, C), jnp.float32),
    )(rbf, w1, b1, w2, b2)


def _segment_sum_sc(src_grouped, w_edges, zeros_tab):
    mesh = plsc.VectorSubcoreMesh(core_axis_name="c", subcore_axis_name="s")

    @functools.partial(
        pl.kernel,
        mesh=mesh,
        out_type=jax.ShapeDtypeStruct((_NC, _NPAD, C), jnp.float32),
        scratch_types=[
            pltpu.VMEM((_NCHUNKS, _CHUNK), jnp.int32),
            pltpu.VMEM((2, _CHUNK, C), jnp.float32),
            pltpu.VMEM_SHARED((_NPAD, C), jnp.float32),
            pltpu.SemaphoreType.DMA,
        ],
    )
    def seg(src_hbm, w_hbm, z_hbm, out_hbm, idx_v, rows_v, table_sh, gsem):
        cid = lax.axis_index("c")
        sid = lax.axis_index("s")
        wid = cid * _NS + sid

        pltpu.sync_copy(z_hbm.at[pl.ds(sid * _RPT, _RPT)],
                        table_sh.at[pl.ds(sid * _RPT, _RPT)])
        plsc.subcore_barrier()

        pltpu.sync_copy(src_hbm.at[wid], idx_v)

        ebase = wid * _EPW
        pltpu.async_copy(w_hbm.at[pl.ds(ebase, _CHUNK)], rows_v.at[0], gsem)

        def body(k, carry):
            nxt = k + 1

            @pl.when(nxt < _NCHUNKS)
            def _():
                pltpu.async_copy(w_hbm.at[pl.ds(ebase + nxt * _CHUNK, _CHUNK)],
                                 rows_v.at[nxt % 2], gsem)

            pltpu.make_async_copy(w_hbm.at[pl.ds(0, _CHUNK)],
                                  rows_v.at[k % 2], gsem).wait()
            pltpu.sync_copy(rows_v.at[k % 2], table_sh.at[idx_v.at[k]],
                            add=True)
            return carry

        lax.fori_loop(0, _NCHUNKS, body, 0)
        plsc.subcore_barrier()

        pltpu.sync_copy(table_sh.at[pl.ds(sid * _RPT, _RPT)],
                        out_hbm.at[cid, pl.ds(sid * _RPT, _RPT)])

    return seg(src_grouped, w_edges, zeros_tab)


def _output_body(x_ref, p_ref, w3_ref, w4_ref, b4_ref, w5_ref, b5_ref, out_ref):
    xb = x_ref[...]
    f = jnp.dot(xb, w3_ref[...], preferred_element_type=jnp.float32)
    conv = f * (p_ref[0] + p_ref[1])
    y = _ssp(jnp.dot(conv, w4_ref[...], preferred_element_type=jnp.float32)
             + b4_ref[...])
    v = jnp.dot(y, w5_ref[...], preferred_element_type=jnp.float32) + b5_ref[...]
    out_ref[...] = xb + v


def _node_output(x, partials, w3, w4, b4, w5, b5):
    grid = (N_NODES // _NB,)
    return pl.pallas_call(
        _output_body,
        grid=grid,
        in_specs=[
            pl.BlockSpec((_NB, C), lambda i: (i, 0)),
            pl.BlockSpec((_NC, _NB, C), lambda i: (0, i, 0)),
            pl.BlockSpec((C, C), lambda i: (0, 0)),
            pl.BlockSpec((C, C), lambda i: (0, 0)),
            pl.BlockSpec((1, C), lambda i: (0, 0)),
            pl.BlockSpec((C, C), lambda i: (0, 0)),
            pl.BlockSpec((1, C), lambda i: (0, 0)),
        ],
        out_specs=pl.BlockSpec((_NB, C), lambda i: (i, 0)),
        out_shape=jax.ShapeDtypeStruct((N_NODES, C), jnp.float32),
    )(x, partials, w3, w4, b4, w5, b5)


def kernel(x, rbf, edge_index, weight1, bias1, weight2, bias2, weight3,
           weight4, bias4, weight5, bias5):
    b1 = bias1.reshape(1, C)
    b2 = bias2.reshape(1, C)
    b4 = bias4.reshape(1, C)
    b5 = bias5.reshape(1, C)

    w = _edge_filter(rbf, weight1, b1, weight2, b2)

    src = edge_index[0].astype(jnp.int32).reshape(_NW, _NCHUNKS, _CHUNK)
    zeros_tab = jnp.zeros((_NPAD, C), jnp.float32)
    partials = _segment_sum_sc(src, w, zeros_tab)

    return _node_output(x, partials, weight3, weight4, b4, weight5, b5)


# speedup vs baseline: 3.2995x; 1.0009x over previous
import functools

import jax
import jax.numpy as jnp
from jax import lax
from jax.experimental import pallas as pl
from jax.experimental.pallas import tpu as pltpu
from jax.experimental.pallas import tpu_sc as plsc

N_NODES = 10000
N_EDGES = 320000
C = 128
N_RBF = 200

_LOG2 = 0.6931471805599453

_NC = 2
_NS = 16
_NW = _NC * _NS
_EPW = N_EDGES // _NW
_CHUNK = 80
_NCHUNKS = _EPW // _CHUNK
_NPAD = 10240
_RPT = _NPAD // _NS

_EB = 2560
_NB = 2000


def _ssp(x):
    safe = jnp.minimum(x, 14.0)
    return jnp.where(x < 14.0, jnp.log(1.0 + jnp.exp(safe)), x) - _LOG2


def _filter_body(rbf_ref, w1_ref, b1_ref, w2_ref, b2_ref, out_ref):
    r = jnp.dot(rbf_ref[...], w1_ref[...], preferred_element_type=jnp.float32)
    r = _ssp(r + b1_ref[...])
    w = jnp.dot(r, w2_ref[...], preferred_element_type=jnp.float32)
    out_ref[...] = _ssp(w + b2_ref[...])


def _edge_filter(rbf, w1, b1, w2, b2):
    grid = (N_EDGES // _EB,)
    return pl.pallas_call(
        _filter_body,
        grid=grid,
        in_specs=[
            pl.BlockSpec((_EB, N_RBF), lambda i: (i, 0)),
            pl.BlockSpec((N_RBF, C), lambda i: (0, 0)),
            pl.BlockSpec((1, C), lambda i: (0, 0)),
            pl.BlockSpec((C, C), lambda i: (0, 0)),
            pl.BlockSpec((1, C), lambda i: (0, 0)),
        ],
        out_specs=pl.BlockSpec((_EB, C), lambda i: (i, 0)),
        out_shape=jax.ShapeDtypeStruct((N_EDGES, C), jnp.float32),
    )(rbf, w1, b1, w2, b2)


def _segment_sum_sc(src_grouped, w_edges, zeros_tab):
    mesh = plsc.VectorSubcoreMesh(core_axis_name="c", subcore_axis_name="s")

    @functools.partial(
        pl.kernel,
        mesh=mesh,
        out_type=jax.ShapeDtypeStruct((_NC, _NPAD, C), jnp.float32),
        scratch_types=[
            pltpu.VMEM((_NCHUNKS, _CHUNK), jnp.int32),
            pltpu.VMEM((2, _CHUNK, C), jnp.float32),
            pltpu.VMEM_SHARED((_NPAD, C), jnp.float32),
            pltpu.SemaphoreType.DMA,
        ],
        compiler_params=pltpu.CompilerParams(use_tc_tiling_on_sc=True),
    )
    def seg(src_hbm, w_hbm, z_hbm, out_hbm, idx_v, rows_v, table_sh, gsem):
        cid = lax.axis_index("c")
        sid = lax.axis_index("s")
        wid = cid * _NS + sid

        pltpu.sync_copy(z_hbm.at[pl.ds(sid * _RPT, _RPT)],
                        table_sh.at[pl.ds(sid * _RPT, _RPT)])
        plsc.subcore_barrier()

        pltpu.sync_copy(src_hbm.at[wid], idx_v)

        ebase = wid * _EPW
        pltpu.async_copy(w_hbm.at[pl.ds(ebase, _CHUNK)], rows_v.at[0], gsem)

        def body(k, carry):
            nxt = k + 1

            @pl.when(nxt < _NCHUNKS)
            def _():
                pltpu.async_copy(w_hbm.at[pl.ds(ebase + nxt * _CHUNK, _CHUNK)],
                                 rows_v.at[nxt % 2], gsem)

            pltpu.make_async_copy(w_hbm.at[pl.ds(0, _CHUNK)],
                                  rows_v.at[k % 2], gsem).wait()
            pltpu.sync_copy(rows_v.at[k % 2], table_sh.at[idx_v.at[k]],
                            add=True)
            return carry

        lax.fori_loop(0, _NCHUNKS, body, 0)
        plsc.subcore_barrier()

        pltpu.sync_copy(table_sh.at[pl.ds(sid * _RPT, _RPT)],
                        out_hbm.at[cid, pl.ds(sid * _RPT, _RPT)])

    return seg(src_grouped, w_edges, zeros_tab)


def _output_body(x_ref, p_ref, w3_ref, w4_ref, b4_ref, w5_ref, b5_ref, out_ref):
    xb = x_ref[...]
    f = jnp.dot(xb, w3_ref[...], preferred_element_type=jnp.float32)
    conv = f * (p_ref[0] + p_ref[1])
    y = _ssp(jnp.dot(conv, w4_ref[...], preferred_element_type=jnp.float32)
             + b4_ref[...])
    v = jnp.dot(y, w5_ref[...], preferred_element_type=jnp.float32) + b5_ref[...]
    out_ref[...] = xb + v


def _node_output(x, partials, w3, w4, b4, w5, b5):
    grid = (N_NODES // _NB,)
    return pl.pallas_call(
        _output_body,
        grid=grid,
        in_specs=[
            pl.BlockSpec((_NB, C), lambda i: (i, 0)),
            pl.BlockSpec((_NC, _NB, C), lambda i: (0, i, 0)),
            pl.BlockSpec((C, C), lambda i: (0, 0)),
            pl.BlockSpec((C, C), lambda i: (0, 0)),
            pl.BlockSpec((1, C), lambda i: (0, 0)),
            pl.BlockSpec((C, C), lambda i: (0, 0)),
            pl.BlockSpec((1, C), lambda i: (0, 0)),
        ],
        out_specs=pl.BlockSpec((_NB, C), lambda i: (i, 0)),
        out_shape=jax.ShapeDtypeStruct((N_NODES, C), jnp.float32),
    )(x, partials, w3, w4, b4, w5, b5)


def kernel(x, rbf, edge_index, weight1, bias1, weight2, bias2, weight3,
           weight4, bias4, weight5, bias5):
    b1 = bias1.reshape(1, C)
    b2 = bias2.reshape(1, C)
    b4 = bias4.reshape(1, C)
    b5 = bias5.reshape(1, C)

    w = _edge_filter(rbf, weight1, b1, weight2, b2)

    src = edge_index[0].astype(jnp.int32).reshape(_NW, _NCHUNKS, _CHUNK)
    zeros_tab = jnp.zeros((_NPAD, C), jnp.float32)
    partials = _segment_sum_sc(src, w, zeros_tab)

    return _node_output(x, partials, weight3, weight4, b4, weight5, b5)


# speedup vs baseline: 5.5485x; 1.6816x over previous
import functools

import jax
import jax.numpy as jnp
from jax import lax
from jax.experimental import pallas as pl
from jax.experimental.pallas import tpu as pltpu
from jax.experimental.pallas import tpu_sc as plsc

N_NODES = 10000
N_EDGES = 320000
C = 128
N_RBF = 200

_LOG2 = 0.6931471805599453

_NC = 2
_NS = 16
_NW = _NC * _NS
_EPW = N_EDGES // _NW
_CHUNK = 80
_NCHUNKS = _EPW // _CHUNK
_NPAD = 10240
_RPT = _NPAD // _NS

_EB = 2560
_NB = 2000


def _ssp(x):
    safe = jnp.minimum(x, 14.0)
    return jnp.where(x < 14.0, jnp.log(1.0 + jnp.exp(safe)), x) - _LOG2


def _filter_body(rbf_ref, w1_ref, b1_ref, w2_ref, b2_ref, out_ref):
    r = lax.dot_general(rbf_ref[...], w1_ref[...],
                        dimension_numbers=(((0,), (0,)), ((), ())),
                        preferred_element_type=jnp.float32)
    r = _ssp(r + b1_ref[...])
    w = jnp.dot(r, w2_ref[...], preferred_element_type=jnp.float32)
    out_ref[...] = _ssp(w + b2_ref[...])


def _edge_filter(rbf_t, w1, b1, w2, b2):
    grid = (N_EDGES // _EB,)
    return pl.pallas_call(
        _filter_body,
        grid=grid,
        in_specs=[
            pl.BlockSpec((N_RBF, _EB), lambda i: (0, i)),
            pl.BlockSpec((N_RBF, C), lambda i: (0, 0)),
            pl.BlockSpec((1, C), lambda i: (0, 0)),
            pl.BlockSpec((C, C), lambda i: (0, 0)),
            pl.BlockSpec((1, C), lambda i: (0, 0)),
        ],
        out_specs=pl.BlockSpec((_EB, C), lambda i: (i, 0)),
        out_shape=jax.ShapeDtypeStruct((N_EDGES, C), jnp.float32),
    )(rbf_t, w1, b1, w2, b2)


def _segment_sum_sc(src_grouped, w_edges, zeros_tab):
    mesh = plsc.VectorSubcoreMesh(core_axis_name="c", subcore_axis_name="s")

    @functools.partial(
        pl.kernel,
        mesh=mesh,
        out_type=jax.ShapeDtypeStruct((_NC, _NPAD, C), jnp.float32),
        scratch_types=[
            pltpu.VMEM((_NCHUNKS, _CHUNK), jnp.int32),
            pltpu.VMEM((2, _CHUNK, C), jnp.float32),
            pltpu.VMEM_SHARED((_NPAD, C), jnp.float32),
            pltpu.SemaphoreType.DMA,
        ],
        compiler_params=pltpu.CompilerParams(use_tc_tiling_on_sc=True),
    )
    def seg(src_hbm, w_hbm, z_hbm, out_hbm, idx_v, rows_v, table_sh, gsem):
        cid = lax.axis_index("c")
        sid = lax.axis_index("s")
        wid = cid * _NS + sid

        pltpu.sync_copy(z_hbm.at[pl.ds(sid * _RPT, _RPT)],
                        table_sh.at[pl.ds(sid * _RPT, _RPT)])
        plsc.subcore_barrier()

        pltpu.sync_copy(src_hbm.at[wid], idx_v)

        ebase = wid * _EPW
        pltpu.async_copy(w_hbm.at[pl.ds(ebase, _CHUNK)], rows_v.at[0], gsem)

        def body(k, carry):
            nxt = k + 1

            @pl.when(nxt < _NCHUNKS)
            def _():
                pltpu.async_copy(w_hbm.at[pl.ds(ebase + nxt * _CHUNK, _CHUNK)],
                                 rows_v.at[nxt % 2], gsem)

            pltpu.make_async_copy(w_hbm.at[pl.ds(0, _CHUNK)],
                                  rows_v.at[k % 2], gsem).wait()
            pltpu.sync_copy(rows_v.at[k % 2], table_sh.at[idx_v.at[k]],
                            add=True)
            return carry

        lax.fori_loop(0, _NCHUNKS, body, 0)
        plsc.subcore_barrier()

        pltpu.sync_copy(table_sh.at[pl.ds(sid * _RPT, _RPT)],
                        out_hbm.at[cid, pl.ds(sid * _RPT, _RPT)])

    return seg(src_grouped, w_edges, zeros_tab)


def _output_body(x_ref, p_ref, w3_ref, w4_ref, b4_ref, w5_ref, b5_ref, out_ref):
    xb = x_ref[...]
    f = jnp.dot(xb, w3_ref[...], preferred_element_type=jnp.float32)
    conv = f * (p_ref[0] + p_ref[1])
    y = _ssp(jnp.dot(conv, w4_ref[...], preferred_element_type=jnp.float32)
             + b4_ref[...])
    v = jnp.dot(y, w5_ref[...], preferred_element_type=jnp.float32) + b5_ref[...]
    out_ref[...] = xb + v


def _node_output(x, partials, w3, w4, b4, w5, b5):
    grid = (N_NODES // _NB,)
    return pl.pallas_call(
        _output_body,
        grid=grid,
        in_specs=[
            pl.BlockSpec((_NB, C), lambda i: (i, 0)),
            pl.BlockSpec((_NC, _NB, C), lambda i: (0, i, 0)),
            pl.BlockSpec((C, C), lambda i: (0, 0)),
            pl.BlockSpec((C, C), lambda i: (0, 0)),
            pl.BlockSpec((1, C), lambda i: (0, 0)),
            pl.BlockSpec((C, C), lambda i: (0, 0)),
            pl.BlockSpec((1, C), lambda i: (0, 0)),
        ],
        out_specs=pl.BlockSpec((_NB, C), lambda i: (i, 0)),
        out_shape=jax.ShapeDtypeStruct((N_NODES, C), jnp.float32),
    )(x, partials, w3, w4, b4, w5, b5)


def kernel(x, rbf, edge_index, weight1, bias1, weight2, bias2, weight3,
           weight4, bias4, weight5, bias5):
    b1 = bias1.reshape(1, C)
    b2 = bias2.reshape(1, C)
    b4 = bias4.reshape(1, C)
    b5 = bias5.reshape(1, C)

    w = _edge_filter(rbf.T, weight1, b1, weight2, b2)

    src = edge_index[0].astype(jnp.int32).reshape(_NW, _NCHUNKS, _CHUNK)
    zeros_tab = jnp.zeros((_NPAD, C), jnp.float32)
    partials = _segment_sum_sc(src, w, zeros_tab)

    return _node_output(x, partials, weight3, weight4, b4, weight5, b5)


# speedup vs baseline: 5.8363x; 1.0519x over previous
import functools

import jax
import jax.numpy as jnp
from jax import lax
from jax.experimental import pallas as pl
from jax.experimental.pallas import tpu as pltpu
from jax.experimental.pallas import tpu_sc as plsc

N_NODES = 10000
N_EDGES = 320000
C = 128
N_RBF = 200

_LOG2 = 0.6931471805599453

_NC = 2
_NS = 16
_NW = _NC * _NS
_EPW = N_EDGES // _NW
_CHUNK = 80
_NCHUNKS = _EPW // _CHUNK
_NPAD = 10240
_RPT = _NPAD // _NS

_EB = 2560
_NB = 2000


def _ssp(x):
    safe = jnp.minimum(x, 14.0)
    return jnp.where(x < 14.0, jnp.log(1.0 + jnp.exp(safe)), x) - _LOG2


def _filter_body(rbf_ref, w1_ref, b1_ref, w2_ref, b2_ref, out_ref):
    r = lax.dot_general(rbf_ref[...], w1_ref[...],
                        dimension_numbers=(((0,), (0,)), ((), ())),
                        preferred_element_type=jnp.float32)
    r = _ssp(r + b1_ref[...])
    w = jnp.dot(r, w2_ref[...], preferred_element_type=jnp.float32)
    out_ref[...] = _ssp(w + b2_ref[...])


def _edge_filter(rbf_t, w1, b1, w2, b2):
    grid = (N_EDGES // _EB,)
    return pl.pallas_call(
        _filter_body,
        grid=grid,
        in_specs=[
            pl.BlockSpec((N_RBF, _EB), lambda i: (0, i)),
            pl.BlockSpec((N_RBF, C), lambda i: (0, 0)),
            pl.BlockSpec((1, C), lambda i: (0, 0)),
            pl.BlockSpec((C, C), lambda i: (0, 0)),
            pl.BlockSpec((1, C), lambda i: (0, 0)),
        ],
        out_specs=pl.BlockSpec((_EB, C), lambda i: (i, 0)),
        out_shape=jax.ShapeDtypeStruct((N_EDGES, C), jnp.float32),
    )(rbf_t, w1, b1, w2, b2)


def _segment_sum_sc(src_grouped, w_edges, zeros_tab):
    mesh = plsc.VectorSubcoreMesh(core_axis_name="c", subcore_axis_name="s")

    @functools.partial(
        pl.kernel,
        mesh=mesh,
        out_type=jax.ShapeDtypeStruct((_NC, _NPAD, C), jnp.float32),
        scratch_types=[
            pltpu.VMEM((_NCHUNKS, _CHUNK), jnp.int32),
            pltpu.VMEM((3, _CHUNK, C), jnp.float32),
            pltpu.VMEM_SHARED((_NPAD, C), jnp.float32),
            pltpu.SemaphoreType.DMA,
            pltpu.SemaphoreType.DMA,
        ],
    )
    def seg(src_hbm, w_hbm, z_hbm, out_hbm, idx_v, rows_v, table_sh, gsem,
            ssem):
        cid = lax.axis_index("c")
        sid = lax.axis_index("s")
        wid = cid * _NS + sid

        pltpu.sync_copy(z_hbm.at[pl.ds(sid * _RPT, _RPT)],
                        table_sh.at[pl.ds(sid * _RPT, _RPT)])
        plsc.subcore_barrier()

        pltpu.sync_copy(src_hbm.at[wid], idx_v)

        ebase = wid * _EPW
        for b in range(2):
            if b < _NCHUNKS:
                pltpu.async_copy(w_hbm.at[pl.ds(ebase + b * _CHUNK, _CHUNK)],
                                 rows_v.at[b], gsem)

        def body(k, carry):
            pltpu.make_async_copy(w_hbm.at[pl.ds(0, _CHUNK)],
                                  rows_v.at[k % 3], gsem).wait()
            pltpu.async_copy(rows_v.at[k % 3], table_sh.at[idx_v.at[k]],
                             ssem, add=True)

            @pl.when(k >= 1)
            def _():
                pltpu.make_async_copy(w_hbm.at[pl.ds(0, _CHUNK)],
                                      rows_v.at[k % 3], ssem).wait()

            @pl.when(k + 2 < _NCHUNKS)
            def _():
                pltpu.async_copy(
                    w_hbm.at[pl.ds(ebase + (k + 2) * _CHUNK, _CHUNK)],
                    rows_v.at[(k + 2) % 3], gsem)

            return carry

        lax.fori_loop(0, _NCHUNKS, body, 0)
        pltpu.make_async_copy(w_hbm.at[pl.ds(0, _CHUNK)], rows_v.at[0],
                              ssem).wait()
        plsc.subcore_barrier()

        pltpu.sync_copy(table_sh.at[pl.ds(sid * _RPT, _RPT)],
                        out_hbm.at[cid, pl.ds(sid * _RPT, _RPT)])

    return seg(src_grouped, w_edges, zeros_tab)


def _output_body(x_ref, p_ref, w3_ref, w4_ref, b4_ref, w5_ref, b5_ref, out_ref):
    xb = x_ref[...]
    f = jnp.dot(xb, w3_ref[...], preferred_element_type=jnp.float32)
    conv = f * (p_ref[0] + p_ref[1])
    y = _ssp(jnp.dot(conv, w4_ref[...], preferred_element_type=jnp.float32)
             + b4_ref[...])
    v = jnp.dot(y, w5_ref[...], preferred_element_type=jnp.float32) + b5_ref[...]
    out_ref[...] = xb + v


def _node_output(x, partials, w3, w4, b4, w5, b5):
    grid = (N_NODES // _NB,)
    return pl.pallas_call(
        _output_body,
        grid=grid,
        in_specs=[
            pl.BlockSpec((_NB, C), lambda i: (i, 0)),
            pl.BlockSpec((_NC, _NB, C), lambda i: (0, i, 0)),
            pl.BlockSpec((C, C), lambda i: (0, 0)),
            pl.BlockSpec((C, C), lambda i: (0, 0)),
            pl.BlockSpec((1, C), lambda i: (0, 0)),
            pl.BlockSpec((C, C), lambda i: (0, 0)),
            pl.BlockSpec((1, C), lambda i: (0, 0)),
        ],
        out_specs=pl.BlockSpec((_NB, C), lambda i: (i, 0)),
        out_shape=jax.ShapeDtypeStruct((N_NODES, C), jnp.float32),
    )(x, partials, w3, w4, b4, w5, b5)


def kernel(x, rbf, edge_index, weight1, bias1, weight2, bias2, weight3,
           weight4, bias4, weight5, bias5):
    b1 = bias1.reshape(1, C)
    b2 = bias2.reshape(1, C)
    b4 = bias4.reshape(1, C)
    b5 = bias5.reshape(1, C)

    w = _edge_filter(rbf.T, weight1, b1, weight2, b2)

    src = edge_index[0].astype(jnp.int32).reshape(_NW, _NCHUNKS, _CHUNK)
    zeros_tab = jnp.zeros((_NPAD, C), jnp.float32)
    partials = _segment_sum_sc(src, w, zeros_tab)

    return _node_output(x, partials, weight3, weight4, b4, weight5, b5)


# speedup vs baseline: 6.7993x; 1.1650x over previous
import functools

import jax
import jax.numpy as jnp
from jax import lax
from jax.experimental import pallas as pl
from jax.experimental.pallas import tpu as pltpu
from jax.experimental.pallas import tpu_sc as plsc

N_NODES = 10000
N_EDGES = 320000
C = 128
N_RBF = 200

_LOG2 = 0.6931471805599453

_NC = 2
_NS = 16
_NW = _NC * _NS
_EPW = N_EDGES // _NW
_CHUNK = 80
_NCHUNKS = _EPW // _CHUNK
_NPAD = 10240
_RPT = _NPAD // _NS

_EB = 6400
_NB = 2000


def _ssp(x):
    safe = jnp.minimum(x, 14.0)
    return jnp.where(x < 14.0, jnp.log(1.0 + jnp.exp(safe)), x) - _LOG2


def _filter_body(rbf_ref, w1_ref, b1_ref, w2_ref, b2_ref, out_ref):
    r = lax.dot_general(rbf_ref[...], w1_ref[...],
                        dimension_numbers=(((0,), (0,)), ((), ())),
                        preferred_element_type=jnp.float32)
    r = _ssp(r + b1_ref[...])
    w = jnp.dot(r, w2_ref[...], preferred_element_type=jnp.float32)
    out_ref[...] = _ssp(w + b2_ref[...])


def _edge_filter(rbf_t, w1, b1, w2, b2):
    grid = (N_EDGES // _EB,)
    return pl.pallas_call(
        _filter_body,
        grid=grid,
        in_specs=[
            pl.BlockSpec((N_RBF, _EB), lambda i: (0, i)),
            pl.BlockSpec((N_RBF, C), lambda i: (0, 0)),
            pl.BlockSpec((1, C), lambda i: (0, 0)),
            pl.BlockSpec((C, C), lambda i: (0, 0)),
            pl.BlockSpec((1, C), lambda i: (0, 0)),
        ],
        out_specs=pl.BlockSpec((_EB, C), lambda i: (i, 0)),
        out_shape=jax.ShapeDtypeStruct((N_EDGES, C), jnp.float32),
    )(rbf_t, w1, b1, w2, b2)


def _segment_sum_sc(src_grouped, w_edges, zeros_tab):
    mesh = plsc.VectorSubcoreMesh(core_axis_name="c", subcore_axis_name="s")

    @functools.partial(
        pl.kernel,
        mesh=mesh,
        out_type=jax.ShapeDtypeStruct((_NC, _NPAD, C), jnp.float32),
        scratch_types=[
            pltpu.VMEM((_NCHUNKS, _CHUNK), jnp.int32),
            pltpu.VMEM((3, _CHUNK, C), jnp.float32),
            pltpu.VMEM_SHARED((_NPAD, C), jnp.float32),
            pltpu.SemaphoreType.DMA,
            pltpu.SemaphoreType.DMA,
        ],
    )
    def seg(src_hbm, w_hbm, z_hbm, out_hbm, idx_v, rows_v, table_sh, gsem,
            ssem):
        cid = lax.axis_index("c")
        sid = lax.axis_index("s")
        wid = cid * _NS + sid

        pltpu.sync_copy(z_hbm.at[pl.ds(sid * _RPT, _RPT)],
                        table_sh.at[pl.ds(sid * _RPT, _RPT)])
        plsc.subcore_barrier()

        pltpu.sync_copy(src_hbm.at[wid], idx_v)

        ebase = wid * _EPW
        for b in range(2):
            if b < _NCHUNKS:
                pltpu.async_copy(w_hbm.at[pl.ds(ebase + b * _CHUNK, _CHUNK)],
                                 rows_v.at[b], gsem)

        def body(k, carry):
            pltpu.make_async_copy(w_hbm.at[pl.ds(0, _CHUNK)],
                                  rows_v.at[k % 3], gsem).wait()
            pltpu.async_copy(rows_v.at[k % 3], table_sh.at[idx_v.at[k]],
                             ssem, add=True)

            @pl.when(k >= 1)
            def _():
                pltpu.make_async_copy(w_hbm.at[pl.ds(0, _CHUNK)],
                                      rows_v.at[k % 3], ssem).wait()

            @pl.when(k + 2 < _NCHUNKS)
            def _():
                pltpu.async_copy(
                    w_hbm.at[pl.ds(ebase + (k + 2) * _CHUNK, _CHUNK)],
                    rows_v.at[(k + 2) % 3], gsem)

            return carry

        lax.fori_loop(0, _NCHUNKS, body, 0)
        pltpu.make_async_copy(w_hbm.at[pl.ds(0, _CHUNK)], rows_v.at[0],
                              ssem).wait()
        plsc.subcore_barrier()

        pltpu.sync_copy(table_sh.at[pl.ds(sid * _RPT, _RPT)],
                        out_hbm.at[cid, pl.ds(sid * _RPT, _RPT)])

    return seg(src_grouped, w_edges, zeros_tab)


def _output_body(x_ref, p_ref, w3_ref, w4_ref, b4_ref, w5_ref, b5_ref, out_ref):
    xb = x_ref[...]
    f = jnp.dot(xb, w3_ref[...], preferred_element_type=jnp.float32)
    conv = f * (p_ref[0] + p_ref[1])
    y = _ssp(jnp.dot(conv, w4_ref[...], preferred_element_type=jnp.float32)
             + b4_ref[...])
    v = jnp.dot(y, w5_ref[...], preferred_element_type=jnp.float32) + b5_ref[...]
    out_ref[...] = xb + v


def _node_output(x, partials, w3, w4, b4, w5, b5):
    grid = (N_NODES // _NB,)
    return pl.pallas_call(
        _output_body,
        grid=grid,
        in_specs=[
            pl.BlockSpec((_NB, C), lambda i: (i, 0)),
            pl.BlockSpec((_NC, _NB, C), lambda i: (0, i, 0)),
            pl.BlockSpec((C, C), lambda i: (0, 0)),
            pl.BlockSpec((C, C), lambda i: (0, 0)),
            pl.BlockSpec((1, C), lambda i: (0, 0)),
            pl.BlockSpec((C, C), lambda i: (0, 0)),
            pl.BlockSpec((1, C), lambda i: (0, 0)),
        ],
        out_specs=pl.BlockSpec((_NB, C), lambda i: (i, 0)),
        out_shape=jax.ShapeDtypeStruct((N_NODES, C), jnp.float32),
    )(x, partials, w3, w4, b4, w5, b5)


def kernel(x, rbf, edge_index, weight1, bias1, weight2, bias2, weight3,
           weight4, bias4, weight5, bias5):
    b1 = bias1.reshape(1, C)
    b2 = bias2.reshape(1, C)
    b4 = bias4.reshape(1, C)
    b5 = bias5.reshape(1, C)

    w = _edge_filter(rbf.T, weight1, b1, weight2, b2)

    src = edge_index[0].astype(jnp.int32).reshape(_NW, _NCHUNKS, _CHUNK)
    zeros_tab = jnp.zeros((_NPAD, C), jnp.float32)
    partials = _segment_sum_sc(src, w, zeros_tab)

    return _node_output(x, partials, weight3, weight4, b4, weight5, b5)


# speedup vs baseline: 7.1828x; 1.0564x over previous
import functools

import jax
import jax.numpy as jnp
from jax import lax
from jax.experimental import pallas as pl
from jax.experimental.pallas import tpu as pltpu
from jax.experimental.pallas import tpu_sc as plsc

N_NODES = 10000
N_EDGES = 320000
C = 128
N_RBF = 200

_LOG2 = 0.6931471805599453

_NC = 2
_NS = 16
_NW = _NC * _NS
_EPW = N_EDGES // _NW
_CHUNK = 80
_NCHUNKS = _EPW // _CHUNK
_NPAD = 10240
_RPT = _NPAD // _NS

_EB = 12800
_NB = 2000


def _ssp(x):
    safe = jnp.minimum(x, 14.0)
    return jnp.where(x < 14.0, jnp.log(1.0 + jnp.exp(safe)), x) - _LOG2


def _filter_body(rbf_ref, w1_ref, b1_ref, w2_ref, b2_ref, out_ref):
    r = lax.dot_general(rbf_ref[...], w1_ref[...],
                        dimension_numbers=(((0,), (0,)), ((), ())),
                        preferred_element_type=jnp.float32)
    r = _ssp(r + b1_ref[...])
    w = jnp.dot(r, w2_ref[...], preferred_element_type=jnp.float32)
    out_ref[...] = _ssp(w + b2_ref[...])


def _edge_filter(rbf_t, w1, b1, w2, b2):
    grid = (N_EDGES // _EB,)
    return pl.pallas_call(
        _filter_body,
        grid=grid,
        in_specs=[
            pl.BlockSpec((N_RBF, _EB), lambda i: (0, i)),
            pl.BlockSpec((N_RBF, C), lambda i: (0, 0)),
            pl.BlockSpec((1, C), lambda i: (0, 0)),
            pl.BlockSpec((C, C), lambda i: (0, 0)),
            pl.BlockSpec((1, C), lambda i: (0, 0)),
        ],
        out_specs=pl.BlockSpec((_EB, C), lambda i: (i, 0)),
        out_shape=jax.ShapeDtypeStruct((N_EDGES, C), jnp.float32),
    )(rbf_t, w1, b1, w2, b2)


def _segment_sum_sc(src_grouped, w_edges, zeros_tab):
    mesh = plsc.VectorSubcoreMesh(core_axis_name="c", subcore_axis_name="s")

    @functools.partial(
        pl.kernel,
        mesh=mesh,
        out_type=jax.ShapeDtypeStruct((_NC, _NPAD, C), jnp.float32),
        scratch_types=[
            pltpu.VMEM((_NCHUNKS, _CHUNK), jnp.int32),
            pltpu.VMEM((3, _CHUNK, C), jnp.float32),
            pltpu.VMEM_SHARED((_NPAD, C), jnp.float32),
            pltpu.SemaphoreType.DMA,
            pltpu.SemaphoreType.DMA,
        ],
    )
    def seg(src_hbm, w_hbm, z_hbm, out_hbm, idx_v, rows_v, table_sh, gsem,
            ssem):
        cid = lax.axis_index("c")
        sid = lax.axis_index("s")
        wid = cid * _NS + sid

        pltpu.sync_copy(z_hbm.at[pl.ds(sid * _RPT, _RPT)],
                        table_sh.at[pl.ds(sid * _RPT, _RPT)])
        plsc.subcore_barrier()

        pltpu.sync_copy(src_hbm.at[wid], idx_v)

        ebase = wid * _EPW
        for b in range(2):
            if b < _NCHUNKS:
                pltpu.async_copy(w_hbm.at[pl.ds(ebase + b * _CHUNK, _CHUNK)],
                                 rows_v.at[b], gsem)

        def body(k, carry):
            pltpu.make_async_copy(w_hbm.at[pl.ds(0, _CHUNK)],
                                  rows_v.at[k % 3], gsem).wait()
            pltpu.async_copy(rows_v.at[k % 3], table_sh.at[idx_v.at[k]],
                             ssem, add=True)

            @pl.when(k >= 1)
            def _():
                pltpu.make_async_copy(w_hbm.at[pl.ds(0, _CHUNK)],
                                      rows_v.at[k % 3], ssem).wait()

            @pl.when(k + 2 < _NCHUNKS)
            def _():
                pltpu.async_copy(
                    w_hbm.at[pl.ds(ebase + (k + 2) * _CHUNK, _CHUNK)],
                    rows_v.at[(k + 2) % 3], gsem)

            return carry

        lax.fori_loop(0, _NCHUNKS, body, 0)
        pltpu.make_async_copy(w_hbm.at[pl.ds(0, _CHUNK)], rows_v.at[0],
                              ssem).wait()
        plsc.subcore_barrier()

        pltpu.sync_copy(table_sh.at[pl.ds(sid * _RPT, _RPT)],
                        out_hbm.at[cid, pl.ds(sid * _RPT, _RPT)])

    return seg(src_grouped, w_edges, zeros_tab)


def _output_body(x_ref, p_ref, w3_ref, w4_ref, b4_ref, w5_ref, b5_ref, out_ref):
    xb = x_ref[...]
    f = jnp.dot(xb, w3_ref[...], preferred_element_type=jnp.float32)
    conv = f * (p_ref[0] + p_ref[1])
    y = _ssp(jnp.dot(conv, w4_ref[...], preferred_element_type=jnp.float32)
             + b4_ref[...])
    v = jnp.dot(y, w5_ref[...], preferred_element_type=jnp.float32) + b5_ref[...]
    out_ref[...] = xb + v


def _node_output(x, partials, w3, w4, b4, w5, b5):
    grid = (N_NODES // _NB,)
    return pl.pallas_call(
        _output_body,
        grid=grid,
        in_specs=[
            pl.BlockSpec((_NB, C), lambda i: (i, 0)),
            pl.BlockSpec((_NC, _NB, C), lambda i: (0, i, 0)),
            pl.BlockSpec((C, C), lambda i: (0, 0)),
            pl.BlockSpec((C, C), lambda i: (0, 0)),
            pl.BlockSpec((1, C), lambda i: (0, 0)),
            pl.BlockSpec((C, C), lambda i: (0, 0)),
            pl.BlockSpec((1, C), lambda i: (0, 0)),
        ],
        out_specs=pl.BlockSpec((_NB, C), lambda i: (i, 0)),
        out_shape=jax.ShapeDtypeStruct((N_NODES, C), jnp.float32),
    )(x, partials, w3, w4, b4, w5, b5)


def kernel(x, rbf, edge_index, weight1, bias1, weight2, bias2, weight3,
           weight4, bias4, weight5, bias5):
    b1 = bias1.reshape(1, C)
    b2 = bias2.reshape(1, C)
    b4 = bias4.reshape(1, C)
    b5 = bias5.reshape(1, C)

    w = _edge_filter(rbf.T, weight1, b1, weight2, b2)

    src = edge_index[0].astype(jnp.int32).reshape(_NW, _NCHUNKS, _CHUNK)
    zeros_tab = jnp.zeros((_NPAD, C), jnp.float32)
    partials = _segment_sum_sc(src, w, zeros_tab)

    return _node_output(x, partials, weight3, weight4, b4, weight5, b5)


# speedup vs baseline: 7.2481x; 1.0091x over previous
import functools

import jax
import jax.numpy as jnp
from jax import lax
from jax.experimental import pallas as pl
from jax.experimental.pallas import tpu as pltpu
from jax.experimental.pallas import tpu_sc as plsc

N_NODES = 10000
N_EDGES = 320000
C = 128
N_RBF = 200

_LOG2 = 0.6931471805599453

_NC = 2
_NS = 16
_NW = _NC * _NS
_EPW = N_EDGES // _NW
_CHUNK = 80
_NCHUNKS = _EPW // _CHUNK
_NPAD = 10240
_RPT = _NPAD // _NS

_EB = 16000
_NB = 2000


def _ssp(x):
    safe = jnp.minimum(x, 14.0)
    return jnp.where(x < 14.0, jnp.log(1.0 + jnp.exp(safe)), x) - _LOG2


def _filter_body(rbf_ref, w1_ref, b1_ref, w2_ref, b2_ref, out_ref):
    r = lax.dot_general(rbf_ref[...], w1_ref[...],
                        dimension_numbers=(((0,), (0,)), ((), ())),
                        preferred_element_type=jnp.float32)
    r = _ssp(r + b1_ref[...])
    w = jnp.dot(r, w2_ref[...], preferred_element_type=jnp.float32)
    out_ref[...] = _ssp(w + b2_ref[...])


def _edge_filter(rbf_t, w1, b1, w2, b2):
    grid = (N_EDGES // _EB,)
    return pl.pallas_call(
        _filter_body,
        grid=grid,
        in_specs=[
            pl.BlockSpec((N_RBF, _EB), lambda i: (0, i)),
            pl.BlockSpec((N_RBF, C), lambda i: (0, 0)),
            pl.BlockSpec((1, C), lambda i: (0, 0)),
            pl.BlockSpec((C, C), lambda i: (0, 0)),
            pl.BlockSpec((1, C), lambda i: (0, 0)),
        ],
        out_specs=pl.BlockSpec((_EB, C), lambda i: (i, 0)),
        out_shape=jax.ShapeDtypeStruct((N_EDGES, C), jnp.float32),
    )(rbf_t, w1, b1, w2, b2)


def _segment_sum_sc(src_grouped, w_edges, zeros_tab):
    mesh = plsc.VectorSubcoreMesh(core_axis_name="c", subcore_axis_name="s")

    @functools.partial(
        pl.kernel,
        mesh=mesh,
        out_type=jax.ShapeDtypeStruct((_NC, _NPAD, C), jnp.float32),
        scratch_types=[
            pltpu.VMEM((_NCHUNKS, _CHUNK), jnp.int32),
            pltpu.VMEM((3, _CHUNK, C), jnp.float32),
            pltpu.VMEM_SHARED((_NPAD, C), jnp.float32),
            pltpu.SemaphoreType.DMA,
            pltpu.SemaphoreType.DMA,
        ],
    )
    def seg(src_hbm, w_hbm, z_hbm, out_hbm, idx_v, rows_v, table_sh, gsem,
            ssem):
        cid = lax.axis_index("c")
        sid = lax.axis_index("s")
        wid = cid * _NS + sid

        pltpu.sync_copy(z_hbm.at[pl.ds(sid * _RPT, _RPT)],
                        table_sh.at[pl.ds(sid * _RPT, _RPT)])
        plsc.subcore_barrier()

        pltpu.sync_copy(src_hbm.at[wid], idx_v)

        ebase = wid * _EPW
        for b in range(2):
            if b < _NCHUNKS:
                pltpu.async_copy(w_hbm.at[pl.ds(ebase + b * _CHUNK, _CHUNK)],
                                 rows_v.at[b], gsem)

        def body(k, carry):
            pltpu.make_async_copy(w_hbm.at[pl.ds(0, _CHUNK)],
                                  rows_v.at[k % 3], gsem).wait()
            pltpu.async_copy(rows_v.at[k % 3], table_sh.at[idx_v.at[k]],
                             ssem, add=True)

            @pl.when(k >= 1)
            def _():
                pltpu.make_async_copy(w_hbm.at[pl.ds(0, _CHUNK)],
                                      rows_v.at[k % 3], ssem).wait()

            @pl.when(k + 2 < _NCHUNKS)
            def _():
                pltpu.async_copy(
                    w_hbm.at[pl.ds(ebase + (k + 2) * _CHUNK, _CHUNK)],
                    rows_v.at[(k + 2) % 3], gsem)

            return carry

        lax.fori_loop(0, _NCHUNKS, body, 0)
        pltpu.make_async_copy(w_hbm.at[pl.ds(0, _CHUNK)], rows_v.at[0],
                              ssem).wait()
        plsc.subcore_barrier()

        pltpu.sync_copy(table_sh.at[pl.ds(sid * _RPT, _RPT)],
                        out_hbm.at[cid, pl.ds(sid * _RPT, _RPT)])

    return seg(src_grouped, w_edges, zeros_tab)


def _output_body(x_ref, p_ref, w3_ref, w4_ref, b4_ref, w5_ref, b5_ref, out_ref):
    xb = x_ref[...]
    f = jnp.dot(xb, w3_ref[...], preferred_element_type=jnp.float32)
    conv = f * (p_ref[0] + p_ref[1])
    y = _ssp(jnp.dot(conv, w4_ref[...], preferred_element_type=jnp.float32)
             + b4_ref[...])
    v = jnp.dot(y, w5_ref[...], preferred_element_type=jnp.float32) + b5_ref[...]
    out_ref[...] = xb + v


def _node_output(x, partials, w3, w4, b4, w5, b5):
    grid = (N_NODES // _NB,)
    return pl.pallas_call(
        _output_body,
        grid=grid,
        in_specs=[
            pl.BlockSpec((_NB, C), lambda i: (i, 0)),
            pl.BlockSpec((_NC, _NB, C), lambda i: (0, i, 0)),
            pl.BlockSpec((C, C), lambda i: (0, 0)),
            pl.BlockSpec((C, C), lambda i: (0, 0)),
            pl.BlockSpec((1, C), lambda i: (0, 0)),
            pl.BlockSpec((C, C), lambda i: (0, 0)),
            pl.BlockSpec((1, C), lambda i: (0, 0)),
        ],
        out_specs=pl.BlockSpec((_NB, C), lambda i: (i, 0)),
        out_shape=jax.ShapeDtypeStruct((N_NODES, C), jnp.float32),
    )(x, partials, w3, w4, b4, w5, b5)


def kernel(x, rbf, edge_index, weight1, bias1, weight2, bias2, weight3,
           weight4, bias4, weight5, bias5):
    b1 = bias1.reshape(1, C)
    b2 = bias2.reshape(1, C)
    b4 = bias4.reshape(1, C)
    b5 = bias5.reshape(1, C)

    w = _edge_filter(rbf.T, weight1, b1, weight2, b2)

    src = edge_index[0].astype(jnp.int32).reshape(_NW, _NCHUNKS, _CHUNK)
    zeros_tab = jnp.zeros((_NPAD, C), jnp.float32)
    partials = _segment_sum_sc(src, w, zeros_tab)

    return _node_output(x, partials, weight3, weight4, b4, weight5, b5)
